# Initial kernel scaffold; baseline (speedup 1.0000x reference)
import jax
import jax.numpy as jnp
from jax.experimental import pallas as pl


def kernel(input, anchors, embs, simplices):
    raise NotImplementedError("write your pallas kernel here")


# speedup vs baseline: 9.9023x; 9.9023x over previous
import dataclasses
import functools

import jax
import jax.numpy as jnp
from jax import lax
from jax.experimental import pallas as pl
from jax.experimental.pallas import tpu as pltpu
from jax.experimental.pallas import tpu_sc as plsc

_W = 128
_L = 16


def _tanh_body(a_ref, o_ref):
    o_ref[...] = jnp.tanh(a_ref[...])


def _tc_tanh(flat2):
    return pl.pallas_call(
        _tanh_body,
        out_shape=jax.ShapeDtypeStruct(flat2.shape, jnp.float32),
    )(flat2)


def _sc_embed(q_t, full, embs, simp_t, n, f):
    mesh = plsc.VectorSubcoreMesh(
        core_axis_name="core", subcore_axis_name="subcore",
        num_cores=2, num_subcores=16,
    )
    cp = pltpu.CompilerParams(use_tc_tiling_on_sc=False)
    if "needs_layout_passes" in pltpu.CompilerParams.__dataclass_fields__:
        cp = dataclasses.replace(cp, needs_layout_passes=False)

    @functools.partial(
        pl.kernel,
        out_type=jax.ShapeDtypeStruct((n, f), jnp.float32),
        mesh=mesh,
        compiler_params=cp,
        scratch_types=[
            pltpu.VMEM((3, _W, 2), jnp.float32),
            pltpu.VMEM((3, _W, f), jnp.float32),
            pltpu.SemaphoreType.DMA,
        ],
    )
    def sc_kernel(q_hbm, full_hbm, embs_hbm, simp_hbm, out_hbm, coords_v, rows_v, sem):
        def body(simp_v, q_v, out_v):
            copies = []
            for j in range(3):
                copies.append(
                    pltpu.async_copy(full_hbm.at[simp_v.at[j]], coords_v.at[j], sem))
                copies.append(
                    pltpu.async_copy(embs_hbm.at[simp_v.at[j]], rows_v.at[j], sem))
            for c in copies:
                c.wait()

            @pl.loop(0, _W, step=_L)
            def _group(b):
                iot = b + lax.iota(jnp.int32, _L)

                def cg(j, c):
                    return plsc.load_gather(
                        coords_v,
                        [jnp.full((_L,), j, jnp.int32), iot,
                         jnp.full((_L,), c, jnp.int32)],
                    )

                v1x, v1y = cg(0, 0), cg(0, 1)
                v2x, v2y = cg(1, 0), cg(1, 1)
                v3x, v3y = cg(2, 0), cg(2, 1)
                x = q_v[0, pl.ds(b, _L)]
                y = q_v[1, pl.ds(b, _L)]
                denom = (v2y - v3y) * (v1x - v3x) + (v3x - v2x) * (v1y - v3y)
                w1v = ((v2y - v3y) * (x - v3x) + (v3x - v2x) * (y - v3y)) / denom
                w2v = ((v3y - v1y) * (x - v3x) + (v1x - v3x) * (y - v3y)) / denom
                w3v = 1.0 - w1v - w2v

                for qi in range(_L):
                    w1 = jnp.full((_L,), w1v[qi])
                    w2 = jnp.full((_L,), w2v[qi])
                    w3 = jnp.full((_L,), w3v[qi])
                    q = b + qi
                    for fb in range(0, f, _L):
                        s = pl.ds(fb, _L)
                        out_v[q, s] = (w1 * rows_v[0, q, s]
                                       + w2 * rows_v[1, q, s]
                                       + w3 * rows_v[2, q, s])

        pltpu.emit_pipeline(
            body,
            grid=(n // _W,),
            in_specs=[
                pl.BlockSpec((3, _W), lambda i: (0, i)),
                pl.BlockSpec((2, _W), lambda i: (0, i)),
            ],
            out_specs=[pl.BlockSpec((_W, f), lambda i: (i, 0))],
            core_axis_name=("core", "subcore"),
            dimension_semantics=(pltpu.PARALLEL,),
        )(simp_hbm, q_hbm, out_hbm)

    return sc_kernel(q_t, full, embs, simp_t)


def kernel(input, anchors, embs, simplices):
    n = input.shape[0]
    p = anchors.shape[0]
    f = embs.shape[1]

    flat = anchors.reshape(-1)
    pad = (-flat.shape[0]) % 128
    flat2 = jnp.pad(flat, (0, pad)).reshape(-1, 128)
    ta = _tc_tanh(flat2).reshape(-1)[: p * 2].reshape(p, 2)
    corners = jnp.array(
        [[-1.0, -1.0], [-1.0, 1.0], [1.0, -1.0], [1.0, 1.0]], dtype=input.dtype
    )
    full = jnp.concatenate([ta, corners], axis=0)

    return _sc_embed(input.T, full, embs, simplices.T, n, f)
